# Initial kernel scaffold; baseline (speedup 1.0000x reference)
import jax
import jax.numpy as jnp
from jax.experimental import pallas as pl


def kernel(t, emb_weight):
    raise NotImplementedError("write your pallas kernel here")


# speedup vs baseline: 1.2062x; 1.2062x over previous
import functools

import jax
import jax.numpy as jnp
from jax import lax
from jax.experimental import pallas as pl
from jax.experimental.pallas import tpu as pltpu
from jax.experimental.pallas import tpu_sc as plsc

MAX_LEN = 240
N_HID = 64
B_TOTAL = 4096 * 200
NC, NS, L = 2, 16, 16
NW = NC * NS
B_PER_W = B_TOTAL // NW
C = 512
G = B_PER_W // C
CHUNKS = C // L


def _sc_gather(idx_flat, table_flat):
    mesh = plsc.VectorSubcoreMesh(
        core_axis_name="c", subcore_axis_name="s",
        num_cores=NC, num_subcores=NS,
    )

    @functools.partial(
        pl.kernel,
        out_type=jax.ShapeDtypeStruct((B_TOTAL * N_HID,), jnp.float32),
        mesh=mesh,
        compiler_params=pltpu.CompilerParams(needs_layout_passes=False),
        scratch_types=[
            pltpu.VMEM((MAX_LEN * N_HID,), jnp.float32),
            pltpu.VMEM((C,), jnp.int32),
            pltpu.VMEM((C * N_HID,), jnp.float32),
            pltpu.SemaphoreType.DMA,
        ],
    )
    def k(idx_hbm, table_hbm, out_hbm, table_v, idx_v, rows_v, sem):
        wid = lax.axis_index("s") * NC + lax.axis_index("c")
        base = wid * B_PER_W
        pltpu.sync_copy(table_hbm, table_v)
        lane = jax.lax.iota(jnp.int32, L)

        @pl.loop(0, G)
        def group(g):
            gbase = base + g * C
            pltpu.sync_copy(idx_hbm.at[pl.ds(gbase, C)], idx_v)

            @pl.loop(0, CHUNKS)
            def chunk(c):
                rows16 = idx_v[pl.ds(c * L, L)]
                src = rows16 * N_HID
                dst = (lane + c * L) * N_HID
                for d in range(N_HID):
                    v = plsc.load_gather(table_v, [src + d])
                    plsc.store_scatter(rows_v, [dst + d], v)

            pltpu.sync_copy(rows_v, out_hbm.at[pl.ds(gbase * N_HID, C * N_HID)])

    return k(idx_flat, table_flat)


def kernel(t, emb_weight):
    idx_flat = t.astype(jnp.int32).reshape(-1)
    out = _sc_gather(idx_flat, emb_weight.reshape(-1))
    return out.reshape(t.shape[0], t.shape[1], N_HID)
